# Initial kernel scaffold; baseline (speedup 1.0000x reference)
import jax
import jax.numpy as jnp
from jax.experimental import pallas as pl


def kernel(x, codebook):
    raise NotImplementedError("write your pallas kernel here")


# speedup vs baseline: 1.5265x; 1.5265x over previous
import functools

import jax
import jax.numpy as jnp
from jax import lax
from jax.experimental import pallas as pl
from jax.experimental.pallas import tpu as pltpu
from jax.experimental.pallas import tpu_sc as plsc

B, T, D, K = 8, 576, 256, 1024


def _dist_body(x_ref, cb_ref, idx_ref, loss_ref):
    i = pl.program_id(0)
    xi = x_ref[0]
    cb = cb_ref[...]
    dot = lax.dot_general(cb, xi, (((1,), (1,)), ((), ())),
                          preferred_element_type=jnp.float32)
    x_sq = jnp.sum(xi * xi, axis=1)
    c_sq = jnp.sum(cb * cb, axis=1)
    dist = c_sq[:, None] + x_sq[None, :] - 2.0 * dot
    minv = jnp.min(dist, axis=1)
    amin = jnp.argmin(dist, axis=1).astype(jnp.int32)
    idx_ref[0] = (amin + i * T)[None, :]

    @pl.when(i == 0)
    def _():
        loss_ref[0, 0] = 0.0

    loss_ref[0, 0] += jnp.sum(minv)


def _distances_argmin(x, codebook):
    return pl.pallas_call(
        _dist_body,
        grid=(B,),
        in_specs=[
            pl.BlockSpec((1, T, D), lambda i: (i, 0, 0)),
            pl.BlockSpec((K, D), lambda i: (0, 0)),
        ],
        out_specs=[
            pl.BlockSpec((1, 1, K), lambda i: (i, 0, 0)),
            pl.BlockSpec(block_shape=(1, 1), index_map=lambda i: (0, 0),
                         memory_space=pltpu.SMEM),
        ],
        out_shape=[
            jax.ShapeDtypeStruct((B, 1, K), jnp.int32),
            jax.ShapeDtypeStruct((1, 1), jnp.float32),
        ],
    )(x, codebook)


_NC, _NS = 2, 16
_NW = _NC * _NS
_BK = B * K
_BPW = _BK // _NW
_CHUNK = 128


def _gather_body(table_hbm, idx_hbm, out_hbm, idx_v, rows_v, sem):
    wid = lax.axis_index("s") * _NC + lax.axis_index("c")
    base = wid * _BPW
    pltpu.sync_copy(idx_hbm.at[pl.ds(base, _BPW)], idx_v)
    copies = []
    for c in range(_BPW // _CHUNK):
        copies.append(pltpu.async_copy(
            table_hbm.at[idx_v.at[pl.ds(c * _CHUNK, _CHUNK)]],
            rows_v.at[pl.ds(c * _CHUNK, _CHUNK)],
            sem))
    for cp in copies:
        cp.wait()
    pltpu.sync_copy(rows_v, out_hbm.at[pl.ds(base, _BPW)])


@functools.cache
def _gather_rows():
    return functools.partial(
        pl.kernel,
        mesh=plsc.VectorSubcoreMesh(core_axis_name="c", subcore_axis_name="s"),
        out_type=jax.ShapeDtypeStruct((_BK, D), jnp.float32),
        scratch_types=[
            pltpu.VMEM((_BPW,), jnp.int32),
            pltpu.VMEM((_BPW, D), jnp.float32),
            pltpu.SemaphoreType.DMA,
        ],
    )(_gather_body)


def kernel(x, codebook):
    idx3, loss_sum = _distances_argmin(x, codebook)
    flat_idx = idx3.reshape(_BK)
    table = x.reshape(B * T, D)
    out = _gather_rows()(table, flat_idx).reshape(B, K, D)
    loss = loss_sum[0, 0] * (2.0 / (B * K * D))
    return out, loss


# speedup vs baseline: 1.5438x; 1.0113x over previous
import functools

import jax
import jax.numpy as jnp
from jax import lax
from jax.experimental import pallas as pl
from jax.experimental.pallas import tpu as pltpu
from jax.experimental.pallas import tpu_sc as plsc

B, T, D, K = 8, 576, 256, 1024


TB = 64
NCH = T // TB
NG = TB // 8


def _dist_body(x_ref, cb_ref, idx_ref, loss_ref, cbn_s, caug_s):
    i = pl.program_id(0)

    @pl.when(i == 0)
    def _():
        cb = cb_ref[...]
        cbn_s[...] = cb * -2.0
        c_sq = jnp.sum(cb * cb, axis=1)
        caug_s[...] = jnp.concatenate(
            [c_sq[:, None], jnp.ones((K, 1), jnp.float32)], axis=1)
        loss_ref[0, 0] = 0.0

    xi = x_ref[0]
    x_sq = jnp.sum(xi * xi, axis=1)
    cbn = cbn_s[...]
    caug = caug_s[...]

    accv = jnp.full((8, K), jnp.inf, jnp.float32)
    acci = jnp.zeros((8, K), jnp.int32)
    for c in range(NCH):
        xi_c = xi[c * TB:(c + 1) * TB, :]
        aug_c = jnp.concatenate(
            [jnp.ones((TB, 1), jnp.float32),
             x_sq[c * TB:(c + 1) * TB, None]], axis=1)
        dc = (lax.dot_general(xi_c, cbn, (((1,), (1,)), ((), ())),
                              preferred_element_type=jnp.float32)
              + lax.dot_general(aug_c, caug, (((1,), (1,)), ((), ())),
                                preferred_element_type=jnp.float32))
        for g in range(NG):
            blk = dc[g * 8:(g + 1) * 8, :]
            upd = blk < accv
            accv = jnp.minimum(accv, blk)
            acci = jnp.where(upd, c * NG + g, acci)

    minv = jnp.min(accv, axis=0)
    tcand = acci * 8 + lax.broadcasted_iota(jnp.int32, (8, K), 0)
    amin = jnp.min(jnp.where(accv == minv[None, :], tcand, T), axis=0)
    idx_ref[0, 0, :] = amin + i * T
    loss_ref[0, 0] += jnp.sum(minv)


def _distances_argmin(x, codebook):
    return pl.pallas_call(
        _dist_body,
        grid=(B,),
        in_specs=[
            pl.BlockSpec((1, T, D), lambda i: (i, 0, 0)),
            pl.BlockSpec((K, D), lambda i: (0, 0)),
        ],
        out_specs=[
            pl.BlockSpec((1, 1, K), lambda i: (i, 0, 0)),
            pl.BlockSpec(block_shape=(1, 1), index_map=lambda i: (0, 0),
                         memory_space=pltpu.SMEM),
        ],
        out_shape=[
            jax.ShapeDtypeStruct((B, 1, K), jnp.int32),
            jax.ShapeDtypeStruct((1, 1), jnp.float32),
        ],
        scratch_shapes=[
            pltpu.VMEM((K, D), jnp.float32),
            pltpu.VMEM((K, 2), jnp.float32),
        ],
    )(x, codebook)


_NC, _NS = 2, 16
_NW = _NC * _NS
_BK = B * K
_BPW = _BK // _NW
_QPB = K // _BPW
_CHUNK = 128


def _gather_body(table_hbm, idx_hbm, out_hbm, idx_v, rows_v, sem):
    wid = lax.axis_index("s") * _NC + lax.axis_index("c")
    base = wid * _BPW
    pltpu.sync_copy(idx_hbm.at[pl.ds(base, _BPW)], idx_v)
    copies = []
    for c in range(_BPW // _CHUNK):
        copies.append(pltpu.async_copy(
            table_hbm.at[idx_v.at[pl.ds(c * _CHUNK, _CHUNK)]],
            rows_v.at[pl.ds(c * _CHUNK, _CHUNK)],
            sem))
    for cp in copies:
        cp.wait()
    pltpu.sync_copy(rows_v, out_hbm.at[pl.ds(base, _BPW)])


@functools.cache
def _gather_rows():
    return functools.partial(
        pl.kernel,
        mesh=plsc.VectorSubcoreMesh(core_axis_name="c", subcore_axis_name="s"),
        out_type=jax.ShapeDtypeStruct((_BK, D), jnp.float32),
        scratch_types=[
            pltpu.VMEM((_BPW,), jnp.int32),
            pltpu.VMEM((_BPW, D), jnp.float32),
            pltpu.SemaphoreType.DMA,
        ],
    )(_gather_body)


def kernel(x, codebook):
    idx4, loss_sum = _distances_argmin(x, codebook)
    flat_idx = idx4.reshape(_BK)
    table = x.reshape(B * T, D)
    out = _gather_rows()(table, flat_idx).reshape(B, K, D)
    loss = loss_sum[0, 0] * (2.0 / (B * K * D))
    return out, loss


# speedup vs baseline: 1.7441x; 1.1297x over previous
import functools

import jax
import jax.numpy as jnp
from jax import lax
from jax.experimental import pallas as pl
from jax.experimental.pallas import tpu as pltpu
from jax.experimental.pallas import tpu_sc as plsc

B, T, D, K = 8, 576, 256, 1024


TB = 64
NCH = T // TB
NG = TB // 8


def _dist_body(x_ref, cb_ref, idx_ref, loss_ref, cbn_s, caug_s):
    i = pl.program_id(0)

    @pl.when(i == 0)
    def _():
        cb = cb_ref[...]
        cbn_s[...] = cb * -2.0
        c_sq = jnp.sum(cb * cb, axis=1)
        caug_s[...] = jnp.broadcast_to(c_sq[None, :], (8, K))
        loss_ref[0, 0] = 0.0

    xi = x_ref[0]
    x_sq = jnp.sum(xi * xi, axis=1)
    cbn = cbn_s[...]
    c_sq8 = caug_s[...]

    accv = jnp.full((8, K), jnp.inf, jnp.float32)
    acci = jnp.zeros((8, K), jnp.int32)
    for c in range(NCH):
        xi_c = xi[c * TB:(c + 1) * TB, :]
        dc = lax.dot_general(xi_c, cbn, (((1,), (1,)), ((), ())),
                             preferred_element_type=jnp.float32)
        for g in range(NG):
            t0 = c * TB + g * 8
            blk = (c_sq8 + x_sq[t0:t0 + 8, None]) + dc[g * 8:(g + 1) * 8, :]
            upd = blk < accv
            accv = jnp.minimum(accv, blk)
            acci = jnp.where(upd, c * NG + g, acci)

    minv = jnp.min(accv, axis=0)
    tcand = acci * 8 + lax.broadcasted_iota(jnp.int32, (8, K), 0)
    amin = jnp.min(jnp.where(accv == minv[None, :], tcand, T), axis=0)
    idx_ref[0, 0, :] = amin + i * T
    loss_ref[0, 0] += jnp.sum(minv)


def _distances_argmin(x, codebook):
    return pl.pallas_call(
        _dist_body,
        grid=(B,),
        in_specs=[
            pl.BlockSpec((1, T, D), lambda i: (i, 0, 0)),
            pl.BlockSpec((K, D), lambda i: (0, 0)),
        ],
        out_specs=[
            pl.BlockSpec((1, 1, K), lambda i: (i, 0, 0)),
            pl.BlockSpec(block_shape=(1, 1), index_map=lambda i: (0, 0),
                         memory_space=pltpu.SMEM),
        ],
        out_shape=[
            jax.ShapeDtypeStruct((B, 1, K), jnp.int32),
            jax.ShapeDtypeStruct((1, 1), jnp.float32),
        ],
        scratch_shapes=[
            pltpu.VMEM((K, D), jnp.float32),
            pltpu.VMEM((8, K), jnp.float32),
        ],
    )(x, codebook)


_NC, _NS = 2, 16
_NW = _NC * _NS
_BK = B * K
_BPW = _BK // _NW
_QPB = K // _BPW
_CHUNK = 128


def _gather_body(table_hbm, idx_hbm, out_hbm, idx_v, rows_v, sem):
    wid = lax.axis_index("s") * _NC + lax.axis_index("c")
    base = wid * _BPW
    pltpu.sync_copy(idx_hbm.at[pl.ds(base, _BPW)], idx_v)
    copies = []
    for c in range(_BPW // _CHUNK):
        copies.append(pltpu.async_copy(
            table_hbm.at[idx_v.at[pl.ds(c * _CHUNK, _CHUNK)]],
            rows_v.at[pl.ds(c * _CHUNK, _CHUNK)],
            sem))
    for cp in copies:
        cp.wait()
    pltpu.sync_copy(rows_v, out_hbm.at[pl.ds(base, _BPW)])


@functools.cache
def _gather_rows():
    return functools.partial(
        pl.kernel,
        mesh=plsc.VectorSubcoreMesh(core_axis_name="c", subcore_axis_name="s"),
        out_type=jax.ShapeDtypeStruct((_BK, D), jnp.float32),
        scratch_types=[
            pltpu.VMEM((_BPW,), jnp.int32),
            pltpu.VMEM((_BPW, D), jnp.float32),
            pltpu.SemaphoreType.DMA,
        ],
    )(_gather_body)


def kernel(x, codebook):
    idx4, loss_sum = _distances_argmin(x, codebook)
    flat_idx = idx4.reshape(_BK)
    table = x.reshape(B * T, D)
    out = _gather_rows()(table, flat_idx).reshape(B, K, D)
    loss = loss_sum[0, 0] * (2.0 / (B * K * D))
    return out, loss
